# baseline (speedup 1.0000x reference)
import functools

import jax
import jax.numpy as jnp
from jax import lax
from jax.experimental import pallas as pl
from jax.experimental.pallas import tpu as pltpu
from jax.experimental.pallas import tpu_sc as plsc

_N_TOKENS = 32768
_D = 768
_E = 64
_TM = 4096

_NC, _NS, _L = 2, 16, 16
_NW = _NC * _NS
_U = 4


def _dense_body(x_ref, w_ref, b_ref, scores_ref):
    x = x_ref[...]
    w = w_ref[...]
    logits = lax.dot_general(
        x, w, (((1,), (1,)), ((), ())), preferred_element_type=jnp.float32
    )
    logits = logits + b_ref[...]
    m = jnp.max(logits, axis=-1, keepdims=True)
    e = jnp.exp(logits - m)
    s = jnp.sum(e, axis=-1, keepdims=True)
    scores_ref[...] = e * (1.0 / s)


def _dense_scores(x, W, b):
    return pl.pallas_call(
        _dense_body,
        grid=(_N_TOKENS // _TM,),
        in_specs=[
            pl.BlockSpec((_TM, _D), lambda i: (i, 0)),
            pl.BlockSpec((_E, _D), lambda i: (0, 0)),
            pl.BlockSpec((1, _E), lambda i: (0, 0)),
        ],
        out_specs=pl.BlockSpec((_TM, _E), lambda i: (i, 0)),
        out_shape=jax.ShapeDtypeStruct((_N_TOKENS, _E), jnp.float32),
    )(x, W, b.reshape(1, _E))


def _make_sc_topk(T):
    tpw = T // _NW
    groups = tpw // _L
    mesh = plsc.VectorSubcoreMesh(
        core_axis_name="c", subcore_axis_name="s",
        num_cores=_NC, num_subcores=_NS,
    )

    @functools.partial(
        pl.kernel,
        out_type=[
            jax.ShapeDtypeStruct((T,), jnp.float32),
            jax.ShapeDtypeStruct((T,), jnp.float32),
            jax.ShapeDtypeStruct((T,), jnp.int32),
            jax.ShapeDtypeStruct((T,), jnp.int32),
        ],
        mesh=mesh,
        compiler_params=pltpu.CompilerParams(needs_layout_passes=False),
        scratch_types=[
            pltpu.VMEM((tpw * _E,), jnp.float32),
            pltpu.VMEM((tpw,), jnp.float32),
            pltpu.VMEM((tpw,), jnp.float32),
            pltpu.VMEM((tpw,), jnp.int32),
            pltpu.VMEM((tpw,), jnp.int32),
        ],
    )
    def sc_topk(scores_hbm, s1_hbm, s2_hbm, i1_hbm, i2_hbm,
                sc_v, s1_v, s2_v, i1_v, i2_v):
        wid = lax.axis_index("s") * _NC + lax.axis_index("c")
        base = wid * tpw
        pltpu.sync_copy(scores_hbm.at[pl.ds(base * _E, tpw * _E)], sc_v)

        lane = lax.broadcasted_iota(jnp.int32, (_L,), 0)

        def group(g, carry):
            flats, k1s, k2s = [], [], []
            for j in range(_U):
                flat = ((g * _U + j) * _L + lane) * _E
                v0 = plsc.load_gather(sc_v, [flat])
                b0 = plsc.bitcast(v0, jnp.int32)
                flats.append(flat)
                k1s.append((b0 & jnp.int32(-64)) | jnp.int32(63))
                k2s.append(jnp.zeros((_L,), jnp.int32))
            for e in range(1, _E):
                for j in range(_U):
                    v = plsc.load_gather(sc_v, [flats[j] + e])
                    bits = plsc.bitcast(v, jnp.int32)
                    nk = (bits & jnp.int32(-64)) | jnp.int32(63 - e)
                    gt = nk > k1s[j]
                    k2s[j] = jnp.where(gt, k1s[j], jnp.maximum(k2s[j], nk))
                    k1s[j] = jnp.maximum(k1s[j], nk)
            for j in range(_U):
                off = (g * _U + j) * _L
                s1_v[pl.ds(off, _L)] = plsc.bitcast(
                    k1s[j] & jnp.int32(-64), jnp.float32)
                s2_v[pl.ds(off, _L)] = plsc.bitcast(
                    k2s[j] & jnp.int32(-64), jnp.float32)
                i1_v[pl.ds(off, _L)] = jnp.int32(63) - (k1s[j] & jnp.int32(63))
                i2_v[pl.ds(off, _L)] = jnp.int32(63) - (k2s[j] & jnp.int32(63))
            return carry

        lax.fori_loop(0, groups // _U, group, 0)

        pltpu.sync_copy(s1_v, s1_hbm.at[pl.ds(base, tpw)])
        pltpu.sync_copy(s2_v, s2_hbm.at[pl.ds(base, tpw)])
        pltpu.sync_copy(i1_v, i1_hbm.at[pl.ds(base, tpw)])
        pltpu.sync_copy(i2_v, i2_hbm.at[pl.ds(base, tpw)])

    return sc_topk


def kernel(x, W, b):
    scores = _dense_scores(x, W, b)
    s1, s2, i1, i2 = _make_sc_topk(_N_TOKENS)(scores.reshape(-1))
    ts = jnp.stack([s1, s2], axis=-1)
    ti = jnp.stack([i1, i2], axis=-1)
    return ts, ti, scores


# speedup vs baseline: 1.2507x; 1.2507x over previous
import functools

import jax
import jax.numpy as jnp
from jax import lax
from jax.experimental import pallas as pl
from jax.experimental.pallas import tpu as pltpu
from jax.experimental.pallas import tpu_sc as plsc

_N_TOKENS = 32768
_D = 768
_E = 64
_TM = 4096

_NC, _NS, _L = 2, 16, 16
_NW = _NC * _NS
_U = 4


def _dense_body(x_ref, w_ref, b_ref, scores_ref):
    x = x_ref[...]
    w = w_ref[...]
    logits = lax.dot_general(
        x, w, (((1,), (1,)), ((), ())), preferred_element_type=jnp.float32
    )
    logits = logits + b_ref[...]
    m = jnp.max(logits, axis=-1, keepdims=True)
    e = jnp.exp(logits - m)
    s = jnp.sum(e, axis=-1, keepdims=True)
    scores_ref[...] = e * (1.0 / s)


def _dense_scores(x, W, b):
    return pl.pallas_call(
        _dense_body,
        grid=(_N_TOKENS // _TM,),
        in_specs=[
            pl.BlockSpec((_TM, _D), lambda i: (i, 0)),
            pl.BlockSpec((_E, _D), lambda i: (0, 0)),
            pl.BlockSpec((1, _E), lambda i: (0, 0)),
        ],
        out_specs=pl.BlockSpec((_TM, _E), lambda i: (i, 0)),
        out_shape=jax.ShapeDtypeStruct((_N_TOKENS, _E), jnp.float32),
    )(x, W, b.reshape(1, _E))


def _make_sc_topk(T):
    tpw = T // _NW
    groups = tpw // _L
    mesh = plsc.VectorSubcoreMesh(
        core_axis_name="c", subcore_axis_name="s",
        num_cores=_NC, num_subcores=_NS,
    )

    @functools.partial(
        pl.kernel,
        out_type=[
            jax.ShapeDtypeStruct((T,), jnp.float32),
            jax.ShapeDtypeStruct((T,), jnp.float32),
            jax.ShapeDtypeStruct((T,), jnp.int32),
            jax.ShapeDtypeStruct((T,), jnp.int32),
        ],
        mesh=mesh,
        compiler_params=pltpu.CompilerParams(needs_layout_passes=False),
        scratch_types=[
            pltpu.VMEM((tpw * _E,), jnp.float32),
            pltpu.VMEM((tpw,), jnp.float32),
            pltpu.VMEM((tpw,), jnp.float32),
            pltpu.VMEM((tpw,), jnp.int32),
            pltpu.VMEM((tpw,), jnp.int32),
        ],
    )
    def sc_topk(scores_hbm, s1_hbm, s2_hbm, i1_hbm, i2_hbm,
                sc_v, s1_v, s2_v, i1_v, i2_v):
        wid = lax.axis_index("s") * _NC + lax.axis_index("c")
        base = wid * tpw
        pltpu.sync_copy(scores_hbm.at[pl.ds(base * _E, tpw * _E)], sc_v)

        lane = lax.broadcasted_iota(jnp.int32, (_L,), 0)

        def group(g, carry):
            flats, k1s, k2s = [], [], []
            rot0 = lane
            key0 = jnp.int32(63) - rot0
            for j in range(_U):
                flat = ((g * _U + j) * _L + lane) * _E
                v0 = plsc.load_gather(sc_v, [flat + rot0])
                b0 = plsc.bitcast(v0, jnp.int32)
                flats.append(flat)
                k1s.append((b0 & jnp.int32(-64)) | key0)
                k2s.append(jnp.zeros((_L,), jnp.int32))
            for e in range(1, _E):
                rot = (lane + jnp.int32(e)) & jnp.int32(63)
                key = jnp.int32(63) - rot
                for j in range(_U):
                    v = plsc.load_gather(sc_v, [flats[j] + rot])
                    bits = plsc.bitcast(v, jnp.int32)
                    nk = (bits & jnp.int32(-64)) | key
                    k2s[j] = jnp.maximum(k2s[j], jnp.minimum(k1s[j], nk))
                    k1s[j] = jnp.maximum(k1s[j], nk)
            for j in range(_U):
                off = (g * _U + j) * _L
                s1_v[pl.ds(off, _L)] = plsc.bitcast(
                    k1s[j] & jnp.int32(-64), jnp.float32)
                s2_v[pl.ds(off, _L)] = plsc.bitcast(
                    k2s[j] & jnp.int32(-64), jnp.float32)
                i1_v[pl.ds(off, _L)] = jnp.int32(63) - (k1s[j] & jnp.int32(63))
                i2_v[pl.ds(off, _L)] = jnp.int32(63) - (k2s[j] & jnp.int32(63))
            return carry

        lax.fori_loop(0, groups // _U, group, 0)

        pltpu.sync_copy(s1_v, s1_hbm.at[pl.ds(base, tpw)])
        pltpu.sync_copy(s2_v, s2_hbm.at[pl.ds(base, tpw)])
        pltpu.sync_copy(i1_v, i1_hbm.at[pl.ds(base, tpw)])
        pltpu.sync_copy(i2_v, i2_hbm.at[pl.ds(base, tpw)])

    return sc_topk


def kernel(x, W, b):
    scores = _dense_scores(x, W, b)
    s1, s2, i1, i2 = _make_sc_topk(_N_TOKENS)(scores.reshape(-1))
    ts = jnp.stack([s1, s2], axis=-1)
    ti = jnp.stack([i1, i2], axis=-1)
    return ts, ti, scores
